# Initial kernel scaffold; baseline (speedup 1.0000x reference)
import jax
import jax.numpy as jnp
from jax.experimental import pallas as pl


def kernel(x, edge_index, edge_weight_norm, W, b):
    raise NotImplementedError("write your pallas kernel here")


# speedup vs baseline: 52.2497x; 52.2497x over previous
import functools

import jax
import jax.numpy as jnp
from jax import lax
from jax.experimental import pallas as pl
from jax.experimental.pallas import tpu as pltpu
from jax.experimental.pallas import tpu_sc as plsc

N = 10000
NPAD = 10240
DIM = 128
E = 320000
TILES = 16
ROW = 128
ROWS = 157
EPAD = TILES * ROWS * ROW
UNROLL = ROW // 16

_MESH = plsc.VectorSubcoreMesh(
    core_axis_name="c", subcore_axis_name="s", num_cores=2, num_subcores=16
)


def _sc_body(src_hbm, dst_hbm, w_hbm, zeros_hbm, out_hbm,
             src_loc, dst_loc, w_loc, msg_loc, u_loc, v_sh):
    c = lax.axis_index("c")
    s = lax.axis_index("s")

    pltpu.sync_copy(src_hbm.at[s], src_loc)
    pltpu.sync_copy(w_hbm.at[s], w_loc)

    @pl.when(c == 0)
    def _core0():
        @pl.when(s == 0)
        def _():
            pltpu.sync_copy(zeros_hbm, v_sh)

        plsc.subcore_barrier()

        def deg_row(j, carry):
            pltpu.sync_copy(w_loc.at[j], v_sh.at[src_loc.at[j]], add=True)
            return carry

        lax.fori_loop(0, ROWS, deg_row, 0)
        plsc.subcore_barrier()

        @pl.when(s == 0)
        def _():
            pltpu.sync_copy(v_sh, out_hbm.at[0])

    @pl.when(c == 1)
    def _core1():
        pltpu.sync_copy(dst_hbm.at[s], dst_loc)

        @pl.when(s == 0)
        def _():
            pltpu.sync_copy(zeros_hbm, v_sh)

        plsc.subcore_barrier()

        inv_n = jnp.float32(1.0 / N)

        def p1_row(j, carry):
            for u in range(UNROLL):
                sl = pl.ds(u * 16, 16)
                msg_loc[j, sl] = w_loc[j, sl] * inv_n
            pltpu.sync_copy(msg_loc.at[j], v_sh.at[dst_loc.at[j]], add=True)
            return carry

        lax.fori_loop(0, ROWS, p1_row, 0)
        plsc.subcore_barrier()

        @pl.when(s == 0)
        def _():
            pltpu.sync_copy(v_sh, out_hbm.at[1])

        plsc.subcore_barrier()

        for k in (2, 3):
            pltpu.sync_copy(out_hbm.at[k - 1], u_loc)

            @pl.when(s == 0)
            def _():
                pltpu.sync_copy(zeros_hbm, v_sh)

            plsc.subcore_barrier()

            def pk_row(j, carry):
                for u in range(UNROLL):
                    sl = pl.ds(u * 16, 16)
                    g = plsc.load_gather(u_loc, [src_loc[j, sl]])
                    msg_loc[j, sl] = g * w_loc[j, sl]
                pltpu.sync_copy(msg_loc.at[j], v_sh.at[dst_loc.at[j]], add=True)
                return carry

            lax.fori_loop(0, ROWS, pk_row, 0)
            plsc.subcore_barrier()

            @pl.when(s == 0)
            def _(k=k):
                pltpu.sync_copy(v_sh, out_hbm.at[k])

            plsc.subcore_barrier()


_sc_passes = pl.kernel(
    _sc_body,
    out_type=jax.ShapeDtypeStruct((4, NPAD), jnp.float32),
    mesh=_MESH,
    scratch_types=[
        pltpu.VMEM((ROWS, ROW), jnp.int32),
        pltpu.VMEM((ROWS, ROW), jnp.int32),
        pltpu.VMEM((ROWS, ROW), jnp.float32),
        pltpu.VMEM((ROWS, ROW), jnp.float32),
        pltpu.VMEM((NPAD,), jnp.float32),
        pltpu.VMEM_SHARED((NPAD,), jnp.float32),
    ],
    compiler_params=pltpu.CompilerParams(needs_layout_passes=False),
)


def _tail_body(f_ref, o_ref):
    f = f_ref[...]
    cols = lax.broadcasted_iota(jnp.int32, (1, NPAD), 1)
    mask = (cols < N).astype(jnp.float32)

    deg = jnp.maximum(f[0:1, :], 1e-8)
    ldeg = jnp.log(deg)

    s1 = jnp.sum(f[1:2, :] * mask)
    s2 = jnp.sum(f[2:3, :] * mask)
    s3 = jnp.sum(f[3:4, :] * mask)
    c1 = jnp.maximum(s1, 1e-8)
    c2 = jnp.maximum(s2, 1e-8 * c1)
    c3 = jnp.maximum(s3, 1e-8 * c2)
    pr = f[3:4, :] / c3

    rows = []
    for xr in (deg, ldeg, pr):
        m = jnp.sum(xr * mask) / N
        d = (xr - m) * mask
        var = jnp.sum(d * d) / (N - 1)
        sd = jnp.sqrt(var)
        rows.append((xr - m) / jnp.maximum(sd, 1e-8))
    rows.append(jnp.zeros((1, NPAD), jnp.float32))
    o_ref[...] = jnp.concatenate(rows, axis=0)


_tail = pl.pallas_call(
    _tail_body,
    out_shape=jax.ShapeDtypeStruct((4, NPAD), jnp.float32),
)

_BLK = 1000


def _affine_body(s_ref, wt_ref, b_ref, o_ref):
    o_ref[...] = (
        jnp.dot(s_ref[...], wt_ref[...], preferred_element_type=jnp.float32)
        + b_ref[...]
    )


_affine = pl.pallas_call(
    _affine_body,
    grid=(N // _BLK,),
    in_specs=[
        pl.BlockSpec((_BLK, 4), lambda i: (i, 0)),
        pl.BlockSpec((4, DIM), lambda i: (0, 0)),
        pl.BlockSpec((1, DIM), lambda i: (0, 0)),
    ],
    out_specs=pl.BlockSpec((_BLK, DIM), lambda i: (i, 0)),
    out_shape=jax.ShapeDtypeStruct((N, DIM), jnp.float32),
)


def kernel(x, edge_index, edge_weight_norm, W, b):
    del x
    src = edge_index[0].astype(jnp.int32)
    dst = edge_index[1].astype(jnp.int32)
    w = edge_weight_norm.astype(jnp.float32)

    npad_edges = EPAD - E
    pad_idx = N + (jnp.arange(npad_edges, dtype=jnp.int32) % (NPAD - N))
    src_p = jnp.concatenate([src, pad_idx]).reshape(TILES, ROWS, ROW)
    dst_p = jnp.concatenate([dst, pad_idx]).reshape(TILES, ROWS, ROW)
    w_p = jnp.concatenate([w, jnp.zeros((npad_edges,), jnp.float32)]
                          ).reshape(TILES, ROWS, ROW)
    zeros = jnp.zeros((NPAD,), jnp.float32)

    feats = _sc_passes(src_p, dst_p, w_p, zeros)
    z = _tail(feats)
    s_mat = z.T[:N]
    wt4 = jnp.concatenate([W.T, jnp.zeros((1, DIM), jnp.float32)], axis=0)
    return _affine(s_mat, wt4, b.reshape(1, DIM))
